# baseline (speedup 1.0000x reference)
import functools
import jax
import jax.numpy as jnp
from jax import lax
from jax.experimental import pallas as pl
from jax.experimental.pallas import tpu as pltpu
from jax.experimental.pallas import tpu_sc as plsc

H = 192
NBINS = 512
ROWS = 1024
NC, NS = 2, 16
NW = NC * NS


def _mlp_kernel(h0_ref, h1_ref, h2_ref, h3_ref, cnt_ref, t2h_ref, t2l_ref,
                w1_ref, b1_ref, lng_ref, lnb_ref, w2_ref, b2_ref,
                w3_ref, b3_ref, wa_ref, ba_ref, wv_ref, bv_ref, out_ref):
    cnt = jnp.maximum(cnt_ref[...], 1.0)
    ddot = lambda a, b: jnp.dot(a, b, preferred_element_type=jnp.float32)

    def xdot(a, b):
        ah = a.astype(jnp.bfloat16)
        al = (a - ah.astype(jnp.float32)).astype(jnp.bfloat16)
        bh = b.astype(jnp.bfloat16)
        return ddot(ah, bh) + ddot(al, bh)

    rows = cnt_ref.shape[0]
    summary = None
    for g, hg_ref in enumerate((h0_ref, h1_ref, h2_ref, h3_ref)):
        hg = hg_ref[...].reshape(rows, 128)
        t2h_g = t2h_ref[g * 128:(g + 1) * 128, :]
        t2l_g = t2l_ref[g * 128:(g + 1) * 128, :]
        hh = hg.astype(jnp.bfloat16)
        hl = (hg - hh.astype(jnp.float32)).astype(jnp.bfloat16)
        part = (ddot(hh, t2h_g) + ddot(hh, t2l_g) + ddot(hl, t2h_g))
        summary = part if summary is None else summary + part
    summary = summary / jnp.sqrt(cnt)

    h = jnp.maximum(xdot(summary, w1_ref[...]) + b1_ref[...], 0.0)
    mu = jnp.mean(h, axis=-1, keepdims=True)
    var = jnp.mean((h - mu) ** 2, axis=-1, keepdims=True)
    h = (h - mu) / jnp.sqrt(var + 1e-5) * lng_ref[...] + lnb_ref[...]
    h = jnp.maximum(xdot(h, w2_ref[...]) + b2_ref[...], 0.0)
    h = jnp.maximum(xdot(h, w3_ref[...]) + b3_ref[...], 0.0)
    logits = xdot(h, wa_ref[...]) + ba_ref[...]
    hh2 = h.astype(jnp.bfloat16)
    hl2 = (h - hh2.astype(jnp.float32)).astype(jnp.bfloat16)
    wvh = wv_ref[...].astype(jnp.bfloat16)
    wvl = (wv_ref[...] - wvh.astype(jnp.float32)).astype(jnp.bfloat16)
    values = (ddot(hh2, wvh) + ddot(hh2, wvl) + ddot(hl2, wvh)
              + bv_ref[...])
    out_ref[...] = jnp.concatenate([logits, values], axis=1)


def _make_sc_hist(B, T):
    rows_per_w = B // NW
    nblk = rows_per_w // 16
    row_words = 3 * T

    mesh = plsc.VectorSubcoreMesh(core_axis_name="c", subcore_axis_name="s")

    @functools.partial(
        pl.kernel,
        mesh=mesh,
        out_type=[
            jax.ShapeDtypeStruct((B * NBINS,), jnp.float32),
            jax.ShapeDtypeStruct((B,), jnp.float32),
        ],
        scratch_types=[
            pltpu.VMEM((2 * 16 * row_words,), jnp.int32),
            pltpu.VMEM((256,), jnp.float32),
            pltpu.VMEM((256,), jnp.float32),
            pltpu.VMEM((2 * 16 * NBINS,), jnp.float32),
            pltpu.VMEM((rows_per_w,), jnp.float32),
            pltpu.SemaphoreType.DMA,
            pltpu.SemaphoreType.DMA,
            pltpu.SemaphoreType.DMA,
            pltpu.SemaphoreType.DMA,
        ],
        compiler_params=pltpu.CompilerParams(needs_layout_passes=False),
    )
    def sc_hist(obs_hbm, fs_hbm, hist_hbm, cnt_hbm,
                obs_v, fs_v, invs_v, hist_v, cnt_v,
                sin0, sin1, sout0, sout1):
        wid = lax.axis_index("s") * NC + lax.axis_index("c")
        pltpu.sync_copy(fs_hbm, fs_v)
        for j in range(16):
            sl = pl.ds(j * 16, 16)
            invs_v[sl] = 1.0 / (fs_v[sl] + 1e-6)

        lane = lax.iota(jnp.int32, 16)
        lane_row = lane * row_words
        lane_tile = (lane // 8) * (8 * NBINS) + (lane % 8) * 128
        base0 = wid * rows_per_w
        sins = [sin0, sin1]
        souts = [sout0, sout1]

        obs_words = 16 * row_words
        hist_words = 16 * NBINS

        def obs_copy(blk, buf):
            return pltpu.make_async_copy(
                obs_hbm.at[pl.ds((base0 + blk * 16) * row_words, obs_words)],
                obs_v.at[pl.ds(buf * obs_words, obs_words)], sins[buf])

        def hist_copy(blk, buf):
            return pltpu.make_async_copy(
                hist_v.at[pl.ds(buf * hist_words, hist_words)],
                hist_hbm.at[pl.ds((base0 + blk * 16) * NBINS, hist_words)],
                souts[buf])

        obs_copy(0, 0).start()

        def process(blk, cur):
            obs_copy(blk, cur).wait()
            @pl.when(blk + 1 < nblk)
            def _():
                obs_copy(blk + 1, 1 - cur).start()
            @pl.when(blk >= 2)
            def _():
                hist_copy(blk - 2, cur).wait()

            obs_off = cur * obs_words
            hist_off = cur * hist_words

            def zbody(j, _):
                hist_v[pl.ds(hist_off + j * 16, 16)] = (
                    jnp.zeros((16,), jnp.float32))
                return 0
            lax.fori_loop(0, NBINS, zbody, 0, unroll=8)

            lane_tile_c = lane_tile + hist_off

            def tbody(t, cnt):
                col = lane_row + (obs_off + 3 * t)
                c = plsc.load_gather(obs_v, [col])
                f = plsc.load_gather(obs_v, [col + 1])
                v = plsc.load_gather(obs_v, [col + 2])
                f = jnp.clip(f, 0, 255)
                isc = plsc.load_gather(invs_v, [f])
                valid = c != 255
                w = jnp.where(valid, v.astype(jnp.float32), 0.0) * isc
                cbin = ((c >> 4) & 15) * 16 + (c & 15)
                fbin = 256 + f
                ca = lane_tile_c + ((cbin >> 7) << 10) + (cbin & 127)
                fa = lane_tile_c + ((fbin >> 7) << 10) + (fbin & 127)
                plsc.addupdate_scatter(hist_v, [ca], w)
                plsc.addupdate_scatter(hist_v, [fa], w)
                return cnt + jnp.where(valid, 1.0, 0.0)

            cnt = lax.fori_loop(0, T, tbody,
                                jnp.zeros((16,), jnp.float32), unroll=4)
            cnt_v[pl.ds(blk * 16, 16)] = cnt
            hist_copy(blk, cur).start()

        def blk_loop(bb, _):
            process(bb * 2, 0)
            process(bb * 2 + 1, 1)
            return 0
        lax.fori_loop(0, nblk // 2, blk_loop, 0)

        hist_copy(nblk - 2, 0).wait()
        hist_copy(nblk - 1, 1).wait()
        pltpu.sync_copy(cnt_v, cnt_hbm.at[pl.ds(base0, rows_per_w)])

    return sc_hist


def kernel(observations, pos_x, pos_y, feat_embed, feature_scale, W1, b1,
           ln_g, ln_b, W2, b2, W3, b3, Wa, ba, Wv, bv):
    B, T, _ = observations.shape
    obs_flat = observations.reshape(-1)

    hist_flat, cnt = _make_sc_hist(B, T)(obs_flat, feature_scale)
    hist3 = hist_flat.reshape(B // 8, 4 * 8, 128)
    cnt2 = cnt.reshape(B, 1)

    pos_xy = (pos_x[:16, None, :] + pos_y[None, :16, :]).reshape(256, H)
    t2 = jnp.concatenate([pos_xy, feat_embed], axis=0)
    t2h = t2.astype(jnp.bfloat16)
    t2l = (t2 - t2h.astype(jnp.float32)).astype(jnp.bfloat16)
    na = Wa.shape[1]
    nout = na + 1

    rows = min(ROWS, B)
    grid = (B // rows,)
    full = lambda shape: pl.BlockSpec(shape, lambda i: (0, 0))

    out = pl.pallas_call(
        _mlp_kernel,
        grid=grid,
        in_specs=[
            pl.BlockSpec((rows // 8, 8, 128), lambda i: (i, 0, 0)),
            pl.BlockSpec((rows // 8, 8, 128), lambda i: (i, 1, 0)),
            pl.BlockSpec((rows // 8, 8, 128), lambda i: (i, 2, 0)),
            pl.BlockSpec((rows // 8, 8, 128), lambda i: (i, 3, 0)),
            pl.BlockSpec((rows, 1), lambda i: (i, 0)),
            full((NBINS, H)), full((NBINS, H)),
            full((H, H)), full((1, H)), full((1, H)), full((1, H)),
            full((H, H)), full((1, H)),
            full((H, H)), full((1, H)),
            full((H, na)), full((1, na)), full((H, 1)), full((1, 1)),
        ],
        out_specs=pl.BlockSpec((rows, nout), lambda i: (i, 0)),
        out_shape=jax.ShapeDtypeStruct((B, nout), jnp.float32),
    )(hist3, hist3, hist3, hist3, cnt2, t2h, t2l,
      W1, b1.reshape(1, H), ln_g.reshape(1, H), ln_b.reshape(1, H),
      W2, b2.reshape(1, H), W3, b3.reshape(1, H),
      Wa, ba.reshape(1, na), Wv, bv.reshape(1, 1))

    l0 = out[:, :9]
    l1 = out[:, 9:19]
    values = out[:, 19:20]
    return (l0, l1, values)


# speedup vs baseline: 30.9726x; 30.9726x over previous
import functools
import jax
import jax.numpy as jnp
from jax import lax
from jax.experimental import pallas as pl
from jax.experimental.pallas import tpu as pltpu
from jax.experimental.pallas import tpu_sc as plsc

H = 192
NBINS = 512
ROWS = 1024
NC, NS = 2, 16
NW = NC * NS


def _mlp_kernel(h0_ref, h1_ref, h2_ref, h3_ref, cnt_ref, t2h_ref, t2l_ref,
                w1_ref, b1_ref, lng_ref, lnb_ref, w2_ref, b2_ref,
                w3_ref, b3_ref, wa_ref, ba_ref, wv_ref, bv_ref, out_ref):
    cnt = jnp.maximum(cnt_ref[...], 1.0)
    ddot = lambda a, b: jnp.dot(a, b, preferred_element_type=jnp.float32)

    def xdot(a, b):
        ah = a.astype(jnp.bfloat16)
        al = (a - ah.astype(jnp.float32)).astype(jnp.bfloat16)
        bh = b.astype(jnp.bfloat16)
        return ddot(ah, bh) + ddot(al, bh)

    rows = cnt_ref.shape[0]
    summary = None
    for g, hg_ref in enumerate((h0_ref, h1_ref, h2_ref, h3_ref)):
        hg = hg_ref[...].reshape(rows, 128)
        t2h_g = t2h_ref[g * 128:(g + 1) * 128, :]
        t2l_g = t2l_ref[g * 128:(g + 1) * 128, :]
        hh = hg.astype(jnp.bfloat16)
        hl = (hg - hh.astype(jnp.float32)).astype(jnp.bfloat16)
        part = (ddot(hh, t2h_g) + ddot(hh, t2l_g) + ddot(hl, t2h_g))
        summary = part if summary is None else summary + part
    summary = summary / jnp.sqrt(cnt)

    h = jnp.maximum(xdot(summary, w1_ref[...]) + b1_ref[...], 0.0)
    mu = jnp.mean(h, axis=-1, keepdims=True)
    var = jnp.mean((h - mu) ** 2, axis=-1, keepdims=True)
    h = (h - mu) / jnp.sqrt(var + 1e-5) * lng_ref[...] + lnb_ref[...]
    h = jnp.maximum(xdot(h, w2_ref[...]) + b2_ref[...], 0.0)
    h = jnp.maximum(xdot(h, w3_ref[...]) + b3_ref[...], 0.0)
    logits = xdot(h, wa_ref[...]) + ba_ref[...]
    hh2 = h.astype(jnp.bfloat16)
    hl2 = (h - hh2.astype(jnp.float32)).astype(jnp.bfloat16)
    wvh = wv_ref[...].astype(jnp.bfloat16)
    wvl = (wv_ref[...] - wvh.astype(jnp.float32)).astype(jnp.bfloat16)
    values = (ddot(hh2, wvh) + ddot(hh2, wvl) + ddot(hl2, wvh)
              + bv_ref[...])
    out_ref[...] = jnp.concatenate([logits, values], axis=1)


def _make_sc_hist(B, T):
    rows_per_w = B // NW
    NSUP = rows_per_w // 128
    TCH = 40
    NCH = T // TCH

    mesh = plsc.VectorSubcoreMesh(core_axis_name="c", subcore_axis_name="s")

    @functools.partial(
        pl.kernel,
        mesh=mesh,
        out_type=[
            jax.ShapeDtypeStruct((B * NBINS,), jnp.float32),
            jax.ShapeDtypeStruct((B,), jnp.float32),
        ],
        scratch_types=[
            pltpu.VMEM((3 * TCH, 128), jnp.int32),
            pltpu.VMEM((256,), jnp.float32),
            pltpu.VMEM((256,), jnp.float32),
            pltpu.VMEM((128 * NBINS,), jnp.float32),
            pltpu.VMEM((rows_per_w,), jnp.float32),
            pltpu.SemaphoreType.DMA,
        ],
        compiler_params=pltpu.CompilerParams(needs_layout_passes=False),
    )
    def sc_hist(ct_hbm, ft_hbm, vt_hbm, fs_hbm, hist_hbm, cnt_hbm,
                obs_v, fs_v, invs_v, hist_v, cnt_v, sout):
        wid = lax.axis_index("s") * NC + lax.axis_index("c")
        pltpu.sync_copy(fs_hbm, fs_v)
        for j in range(16):
            sl = pl.ds(j * 16, 16)
            invs_v[sl] = 1.0 / (fs_v[sl] + 1e-6)
        for j in range(rows_per_w // 16):
            cnt_v[pl.ds(j * 16, 16)] = jnp.zeros((16,), jnp.float32)

        lane = lax.iota(jnp.int32, 16)
        lane_tile = (lane // 8) * (8 * NBINS) + (lane % 8) * 128
        base0 = wid * rows_per_w

        def hist_copy(sup):
            return pltpu.make_async_copy(
                hist_v,
                hist_hbm.at[pl.ds((base0 + sup * 128) * NBINS, 128 * NBINS)],
                sout)

        def sup_body(sup, _):
            bcol = base0 + sup * 128

            def zbody(j, _):
                hist_v[pl.ds(j * 16, 16)] = jnp.zeros((16,), jnp.float32)
                return 0
            lax.fori_loop(0, 128 * NBINS // 16, zbody, 0, unroll=8)

            def ch_body(ch, _):
                t0 = ch * TCH
                pltpu.sync_copy(ct_hbm.at[pl.ds(t0, TCH), pl.ds(bcol, 128)],
                                obs_v.at[pl.ds(0, TCH), :])
                pltpu.sync_copy(ft_hbm.at[pl.ds(t0, TCH), pl.ds(bcol, 128)],
                                obs_v.at[pl.ds(TCH, TCH), :])
                pltpu.sync_copy(vt_hbm.at[pl.ds(t0, TCH), pl.ds(bcol, 128)],
                                obs_v.at[pl.ds(2 * TCH, TCH), :])

                def tbody(t, accs):
                    new_accs = []
                    for sb in range(8):
                        bsl = pl.ds(sb * 16, 16)
                        c = obs_v[t, bsl]
                        f = obs_v[t + TCH, bsl]
                        v = obs_v[t + 2 * TCH, bsl]
                        f = jnp.clip(f, 0, 255)
                        isc = plsc.load_gather(invs_v, [f])
                        valid = c != 255
                        w = jnp.where(valid, v.astype(jnp.float32),
                                      0.0) * isc
                        cbin = ((c >> 4) & 15) * 16 + (c & 15)
                        fbin = 256 + f
                        base_sb = lane_tile + sb * 8192
                        ca = base_sb + ((cbin >> 7) << 10) + (cbin & 127)
                        fa = base_sb + ((fbin >> 7) << 10) + (fbin & 127)
                        plsc.addupdate_scatter(hist_v, [ca], w)
                        plsc.addupdate_scatter(hist_v, [fa], w)
                        new_accs.append(
                            accs[sb] + jnp.where(valid, 1.0, 0.0))
                    return tuple(new_accs)

                accs = lax.fori_loop(
                    0, TCH, tbody,
                    tuple(jnp.zeros((16,), jnp.float32) for _ in range(8)))
                for sb in range(8):
                    csl = pl.ds(sup * 128 + sb * 16, 16)
                    cnt_v[csl] = cnt_v[csl] + accs[sb]
                return 0

            lax.fori_loop(0, NCH, ch_body, 0)
            hist_copy(sup).start()
            hist_copy(sup).wait()
            return 0

        lax.fori_loop(0, NSUP, sup_body, 0)
        pltpu.sync_copy(cnt_v, cnt_hbm.at[pl.ds(base0, rows_per_w)])

    return sc_hist


def kernel(observations, pos_x, pos_y, feat_embed, feature_scale, W1, b1,
           ln_g, ln_b, W2, b2, W3, b3, Wa, ba, Wv, bv):
    B, T, _ = observations.shape
    ct = observations[:, :, 0].T
    ft = observations[:, :, 1].T
    vt = observations[:, :, 2].T

    hist_flat, cnt = _make_sc_hist(B, T)(ct, ft, vt, feature_scale)
    hist3 = hist_flat.reshape(B // 8, 4 * 8, 128)
    cnt2 = cnt.reshape(B, 1)

    pos_xy = (pos_x[:16, None, :] + pos_y[None, :16, :]).reshape(256, H)
    t2 = jnp.concatenate([pos_xy, feat_embed], axis=0)
    t2h = t2.astype(jnp.bfloat16)
    t2l = (t2 - t2h.astype(jnp.float32)).astype(jnp.bfloat16)
    na = Wa.shape[1]
    nout = na + 1

    rows = min(ROWS, B)
    grid = (B // rows,)
    full = lambda shape: pl.BlockSpec(shape, lambda i: (0, 0))

    out = pl.pallas_call(
        _mlp_kernel,
        grid=grid,
        in_specs=[
            pl.BlockSpec((rows // 8, 8, 128), lambda i: (i, 0, 0)),
            pl.BlockSpec((rows // 8, 8, 128), lambda i: (i, 1, 0)),
            pl.BlockSpec((rows // 8, 8, 128), lambda i: (i, 2, 0)),
            pl.BlockSpec((rows // 8, 8, 128), lambda i: (i, 3, 0)),
            pl.BlockSpec((rows, 1), lambda i: (i, 0)),
            full((NBINS, H)), full((NBINS, H)),
            full((H, H)), full((1, H)), full((1, H)), full((1, H)),
            full((H, H)), full((1, H)),
            full((H, H)), full((1, H)),
            full((H, na)), full((1, na)), full((H, 1)), full((1, 1)),
        ],
        out_specs=pl.BlockSpec((rows, nout), lambda i: (i, 0)),
        out_shape=jax.ShapeDtypeStruct((B, nout), jnp.float32),
    )(hist3, hist3, hist3, hist3, cnt2, t2h, t2l,
      W1, b1.reshape(1, H), ln_g.reshape(1, H), ln_b.reshape(1, H),
      W2, b2.reshape(1, H), W3, b3.reshape(1, H),
      Wa, ba.reshape(1, na), Wv, bv.reshape(1, 1))

    l0 = out[:, :9]
    l1 = out[:, 9:19]
    values = out[:, 19:20]
    return (l0, l1, values)


# speedup vs baseline: 33.3150x; 1.0756x over previous
import functools
import jax
import jax.numpy as jnp
from jax import lax
from jax.experimental import pallas as pl
from jax.experimental.pallas import tpu as pltpu
from jax.experimental.pallas import tpu_sc as plsc

H = 192
NBINS = 512
ROWS = 1024
NC, NS = 2, 16
NW = NC * NS


def _mlp_kernel(h0_ref, h1_ref, h2_ref, h3_ref, cnt_ref, t2h_ref, t2l_ref,
                w1_ref, b1_ref, lng_ref, lnb_ref, w2_ref, b2_ref,
                w3_ref, b3_ref, wa_ref, ba_ref, wv_ref, bv_ref, out_ref):
    cnt = jnp.maximum(cnt_ref[...], 1.0)
    ddot = lambda a, b: jnp.dot(a, b, preferred_element_type=jnp.float32)

    def xdot(a, b):
        ah = a.astype(jnp.bfloat16)
        al = (a - ah.astype(jnp.float32)).astype(jnp.bfloat16)
        bh = b.astype(jnp.bfloat16)
        return ddot(ah, bh) + ddot(al, bh)

    rows = cnt_ref.shape[0]
    summary = None
    for g, hg_ref in enumerate((h0_ref, h1_ref, h2_ref, h3_ref)):
        hg = hg_ref[...].reshape(rows, 128)
        t2h_g = t2h_ref[g * 128:(g + 1) * 128, :]
        t2l_g = t2l_ref[g * 128:(g + 1) * 128, :]
        hh = hg.astype(jnp.bfloat16)
        hl = (hg - hh.astype(jnp.float32)).astype(jnp.bfloat16)
        part = (ddot(hh, t2h_g) + ddot(hh, t2l_g) + ddot(hl, t2h_g))
        summary = part if summary is None else summary + part
    summary = summary / jnp.sqrt(cnt)

    h = jnp.maximum(xdot(summary, w1_ref[...]) + b1_ref[...], 0.0)
    mu = jnp.mean(h, axis=-1, keepdims=True)
    var = jnp.mean((h - mu) ** 2, axis=-1, keepdims=True)
    h = (h - mu) / jnp.sqrt(var + 1e-5) * lng_ref[...] + lnb_ref[...]
    h = jnp.maximum(xdot(h, w2_ref[...]) + b2_ref[...], 0.0)
    h = jnp.maximum(xdot(h, w3_ref[...]) + b3_ref[...], 0.0)
    logits = xdot(h, wa_ref[...]) + ba_ref[...]
    hh2 = h.astype(jnp.bfloat16)
    hl2 = (h - hh2.astype(jnp.float32)).astype(jnp.bfloat16)
    wvh = wv_ref[...].astype(jnp.bfloat16)
    wvl = (wv_ref[...] - wvh.astype(jnp.float32)).astype(jnp.bfloat16)
    values = (ddot(hh2, wvh) + ddot(hh2, wvl) + ddot(hl2, wvh)
              + bv_ref[...])
    out_ref[...] = jnp.concatenate([logits, values], axis=1)


def _make_sc_hist(B, T):
    rows_per_w = B // NW
    NSUP = rows_per_w // 128
    TCH = 40
    NCH = T // TCH

    mesh = plsc.VectorSubcoreMesh(core_axis_name="c", subcore_axis_name="s")

    @functools.partial(
        pl.kernel,
        mesh=mesh,
        out_type=[
            jax.ShapeDtypeStruct((B * NBINS,), jnp.float32),
            jax.ShapeDtypeStruct((B,), jnp.float32),
        ],
        scratch_types=[
            pltpu.VMEM((3 * TCH, 128), jnp.int32),
            pltpu.VMEM((256,), jnp.float32),
            pltpu.VMEM((256,), jnp.float32),
            pltpu.VMEM((128 * NBINS,), jnp.float32),
            pltpu.VMEM((rows_per_w,), jnp.float32),
            pltpu.SemaphoreType.DMA,
            pltpu.SemaphoreType.DMA,
            pltpu.SemaphoreType.DMA,
            pltpu.SemaphoreType.DMA,
        ],
        compiler_params=pltpu.CompilerParams(needs_layout_passes=False),
    )
    def sc_hist(ct_hbm, ft_hbm, vt_hbm, fs_hbm, hist_hbm, cnt_hbm,
                obs_v, fs_v, invs_v, hist_v, cnt_v, sout, sc0, sc1, sc2):
        wid = lax.axis_index("s") * NC + lax.axis_index("c")
        pltpu.sync_copy(fs_hbm, fs_v)
        for j in range(16):
            sl = pl.ds(j * 16, 16)
            invs_v[sl] = 1.0 / (fs_v[sl] + 1e-6)
        for j in range(rows_per_w // 16):
            cnt_v[pl.ds(j * 16, 16)] = jnp.zeros((16,), jnp.float32)

        lane = lax.iota(jnp.int32, 16)
        lane_tile = (lane // 8) * (8 * NBINS) + (lane % 8) * 128
        base0 = wid * rows_per_w

        def hist_copy(sup):
            return pltpu.make_async_copy(
                hist_v,
                hist_hbm.at[pl.ds((base0 + sup * 128) * NBINS, 128 * NBINS)],
                sout)

        def sup_body(sup, _):
            bcol = base0 + sup * 128

            def zbody(j, _):
                hist_v[pl.ds(j * 16, 16)] = jnp.zeros((16,), jnp.float32)
                return 0
            lax.fori_loop(0, 128 * NBINS // 16, zbody, 0, unroll=8)

            def ch_body(ch, _):
                t0 = ch * TCH
                cps = [
                    pltpu.make_async_copy(
                        ct_hbm.at[pl.ds(t0, TCH), pl.ds(bcol, 128)],
                        obs_v.at[pl.ds(0, TCH), :], sc0),
                    pltpu.make_async_copy(
                        ft_hbm.at[pl.ds(t0, TCH), pl.ds(bcol, 128)],
                        obs_v.at[pl.ds(TCH, TCH), :], sc1),
                    pltpu.make_async_copy(
                        vt_hbm.at[pl.ds(t0, TCH), pl.ds(bcol, 128)],
                        obs_v.at[pl.ds(2 * TCH, TCH), :], sc2),
                ]
                for cp in cps:
                    cp.start()
                for cp in cps:
                    cp.wait()

                def tbody(t, accs):
                    new_accs = []
                    for sb in range(8):
                        bsl = pl.ds(sb * 16, 16)
                        c = obs_v[t, bsl]
                        f = obs_v[t + TCH, bsl]
                        v = obs_v[t + 2 * TCH, bsl]
                        f = jnp.clip(f, 0, 255)
                        isc = plsc.load_gather(invs_v, [f])
                        valid = c != 255
                        w = jnp.where(valid, v.astype(jnp.float32),
                                      0.0) * isc
                        cbin = ((c >> 4) & 15) * 16 + (c & 15)
                        fbin = 256 + f
                        base_sb = lane_tile + sb * 8192
                        ca = base_sb + ((cbin >> 7) << 10) + (cbin & 127)
                        fa = base_sb + ((fbin >> 7) << 10) + (fbin & 127)
                        plsc.addupdate_scatter(hist_v, [ca], w)
                        plsc.addupdate_scatter(hist_v, [fa], w)
                        new_accs.append(
                            accs[sb] + jnp.where(valid, 1.0, 0.0))
                    return tuple(new_accs)

                accs = lax.fori_loop(
                    0, TCH, tbody,
                    tuple(jnp.zeros((16,), jnp.float32) for _ in range(8)))
                for sb in range(8):
                    csl = pl.ds(sup * 128 + sb * 16, 16)
                    cnt_v[csl] = cnt_v[csl] + accs[sb]
                return 0

            lax.fori_loop(0, NCH, ch_body, 0)
            hist_copy(sup).start()
            hist_copy(sup).wait()
            return 0

        lax.fori_loop(0, NSUP, sup_body, 0)
        pltpu.sync_copy(cnt_v, cnt_hbm.at[pl.ds(base0, rows_per_w)])

    return sc_hist


def kernel(observations, pos_x, pos_y, feat_embed, feature_scale, W1, b1,
           ln_g, ln_b, W2, b2, W3, b3, Wa, ba, Wv, bv):
    B, T, _ = observations.shape
    ct = observations[:, :, 0].T
    ft = observations[:, :, 1].T
    vt = observations[:, :, 2].T

    hist_flat, cnt = _make_sc_hist(B, T)(ct, ft, vt, feature_scale)
    hist3 = hist_flat.reshape(B // 8, 4 * 8, 128)
    cnt2 = cnt.reshape(B, 1)

    pos_xy = (pos_x[:16, None, :] + pos_y[None, :16, :]).reshape(256, H)
    t2 = jnp.concatenate([pos_xy, feat_embed], axis=0)
    t2h = t2.astype(jnp.bfloat16)
    t2l = (t2 - t2h.astype(jnp.float32)).astype(jnp.bfloat16)
    na = Wa.shape[1]
    nout = na + 1

    rows = min(ROWS, B)
    grid = (B // rows,)
    full = lambda shape: pl.BlockSpec(shape, lambda i: (0, 0))

    out = pl.pallas_call(
        _mlp_kernel,
        grid=grid,
        in_specs=[
            pl.BlockSpec((rows // 8, 8, 128), lambda i: (i, 0, 0)),
            pl.BlockSpec((rows // 8, 8, 128), lambda i: (i, 1, 0)),
            pl.BlockSpec((rows // 8, 8, 128), lambda i: (i, 2, 0)),
            pl.BlockSpec((rows // 8, 8, 128), lambda i: (i, 3, 0)),
            pl.BlockSpec((rows, 1), lambda i: (i, 0)),
            full((NBINS, H)), full((NBINS, H)),
            full((H, H)), full((1, H)), full((1, H)), full((1, H)),
            full((H, H)), full((1, H)),
            full((H, H)), full((1, H)),
            full((H, na)), full((1, na)), full((H, 1)), full((1, 1)),
        ],
        out_specs=pl.BlockSpec((rows, nout), lambda i: (i, 0)),
        out_shape=jax.ShapeDtypeStruct((B, nout), jnp.float32),
    )(hist3, hist3, hist3, hist3, cnt2, t2h, t2l,
      W1, b1.reshape(1, H), ln_g.reshape(1, H), ln_b.reshape(1, H),
      W2, b2.reshape(1, H), W3, b3.reshape(1, H),
      Wa, ba.reshape(1, na), Wv, bv.reshape(1, 1))

    l0 = out[:, :9]
    l1 = out[:, 9:19]
    values = out[:, 19:20]
    return (l0, l1, values)
